# baseline (speedup 1.0000x reference)
import jax
import jax.numpy as jnp
from jax.experimental import pallas as pl
from jax.experimental.pallas import tpu as pltpu

B = 16
C = 33
N = C - 1
L = 128
D = 768
K = 16
INV_T = 20.0
EPS = 1e-8


def _nce_body(logits_ref, h_ref, out_ref, pos_ref, sims_ref, sims_t_ref):
    b = pl.program_id(0)
    n = pl.program_id(1)

    blk = h_ref[0, 0]
    mean = jnp.sum(blk, axis=0, keepdims=True) * (1.0 / L)

    @pl.when(n == 0)
    def _store_pos():
        pos_ref[...] = mean

    @pl.when(n > 0)
    def _store_sim():
        p = pos_ref[...]
        dot = jnp.sum(p * mean)
        na = jnp.maximum(jnp.sqrt(jnp.sum(p * p)), EPS)
        nb = jnp.maximum(jnp.sqrt(jnp.sum(mean * mean)), EPS)
        val = dot / (na * nb)
        lane = jax.lax.broadcasted_iota(jnp.int32, (1, N), 1)
        row = sims_ref[pl.ds(b, 1), :]
        sims_ref[pl.ds(b, 1), :] = jnp.where(lane == n - 1, val, row)
        lane_t = jax.lax.broadcasted_iota(jnp.int32, (1, B), 1)
        row_t = sims_t_ref[pl.ds(n - 1, 1), :]
        sims_t_ref[pl.ds(n - 1, 1), :] = jnp.where(lane_t == b, val, row_t)

    @pl.when((b == B - 1) & (n == C - 1))
    def _tail():
        a = logits_ref[...] * INV_T
        ii = jax.lax.broadcasted_iota(jnp.int32, (N, N), 1)
        jj = jax.lax.broadcasted_iota(jnp.int32, (N, N), 0)
        loss_sum = jnp.zeros((1, 1), dtype=jnp.float32)
        for bi in range(B):
            s_lane = jnp.broadcast_to(sims_ref[bi:bi + 1, :], (N, N))
            s_sub = jnp.broadcast_to(sims_t_ref[:, bi:bi + 1], (N, N))
            beats = (s_sub > s_lane) | ((s_sub == s_lane) & (jj < ii))
            rank = jnp.sum(beats.astype(jnp.float32), axis=0, keepdims=True)
            sel = rank < K
            a_row = a[bi:bi + 1, :]
            m = jnp.max(a_row, axis=1, keepdims=True)
            e = jnp.exp(a_row - m)
            z = jnp.sum(e, axis=1, keepdims=True)
            p = e / z
            p0 = p[:, 0:1]
            neg_sum = jnp.sum(jnp.where(sel, p[:, 1:], 0.0),
                              axis=1, keepdims=True)
            loss_sum = loss_sum - jnp.log(p0 / (p0 + neg_sum))
        out_ref[...] = loss_sum * (1.0 / B)


@jax.jit
def kernel(logits, hidden_states):
    out = pl.pallas_call(
        _nce_body,
        grid=(B, C),
        in_specs=[
            pl.BlockSpec((B, C), lambda b, n: (0, 0)),
            pl.BlockSpec((1, 1, L, D), lambda b, n: (b, n, 0, 0)),
        ],
        out_specs=pl.BlockSpec((1, 1), lambda b, n: (0, 0)),
        out_shape=jax.ShapeDtypeStruct((1, 1), jnp.float32),
        scratch_shapes=[
            pltpu.VMEM((1, D), jnp.float32),
            pltpu.VMEM((B, N), jnp.float32),
            pltpu.VMEM((N, B), jnp.float32),
        ],
    )(logits, hidden_states)
    return jnp.reshape(out, ())


# speedup vs baseline: 3.8625x; 3.8625x over previous
import jax
import jax.numpy as jnp
from jax.experimental import pallas as pl
from jax.experimental.pallas import tpu as pltpu

B = 16
C = 33
N = C - 1
L = 128
D = 768
K = 16
CH = 11
NC = C // CH
INV_T = 20.0
EPS = 1e-8


def _tree_sum_rows(x):
    rows = x.shape[0]
    while rows > 8:
        half = rows // 2
        x = x[:half] + x[half:rows]
        rows = half
    return jnp.sum(x, axis=0, keepdims=True)


def _nce_body(logits_ref, h_ref, out_ref, pos_ref, sims_ref, sims_t_ref):
    b = pl.program_id(0)
    c = pl.program_id(1)

    means = [_tree_sum_rows(h_ref[0, i]) * (1.0 / L) for i in range(CH)]

    @pl.when(c == 0)
    def _store_pos():
        pos_ref[...] = means[0]

    p = pos_ref[...]
    na = jnp.maximum(jnp.sqrt(jnp.sum(p * p)), EPS)

    lane_n = jax.lax.broadcasted_iota(jnp.int32, (1, N), 1)
    lane_b = jax.lax.broadcasted_iota(jnp.int32, (N, B), 1)
    sub_n = jax.lax.broadcasted_iota(jnp.int32, (N, B), 0)

    row = sims_ref[pl.ds(b, 1), :]
    tcol = sims_t_ref[...]
    g0 = c * CH
    for i in range(CH):
        m = means[i]
        dot = jnp.sum(p * m)
        nb = jnp.maximum(jnp.sqrt(jnp.sum(m * m)), EPS)
        val = dot / (na * nb)
        g = g0 + i
        row = jnp.where(lane_n == g - 1, val, row)
        tcol = jnp.where((sub_n == g - 1) & (lane_b == b), val, tcol)
    sims_ref[pl.ds(b, 1), :] = row
    sims_t_ref[...] = tcol

    @pl.when((b == B - 1) & (c == NC - 1))
    def _tail():
        a = logits_ref[...] * INV_T
        ii = jax.lax.broadcasted_iota(jnp.int32, (N, N), 1)
        jj = jax.lax.broadcasted_iota(jnp.int32, (N, N), 0)
        loss_sum = jnp.zeros((1, 1), dtype=jnp.float32)
        for bi in range(B):
            s_lane = jnp.broadcast_to(sims_ref[bi:bi + 1, :], (N, N))
            s_sub = jnp.broadcast_to(sims_t_ref[:, bi:bi + 1], (N, N))
            beats = (s_sub > s_lane) | ((s_sub == s_lane) & (jj < ii))
            rank = jnp.sum(beats.astype(jnp.float32), axis=0, keepdims=True)
            sel = rank < K
            a_row = a[bi:bi + 1, :]
            m = jnp.max(a_row, axis=1, keepdims=True)
            e = jnp.exp(a_row - m)
            z = jnp.sum(e, axis=1, keepdims=True)
            pr = e / z
            p0 = pr[:, 0:1]
            neg_sum = jnp.sum(jnp.where(sel, pr[:, 1:], 0.0),
                              axis=1, keepdims=True)
            loss_sum = loss_sum - jnp.log(p0 / (p0 + neg_sum))
        out_ref[...] = loss_sum * (1.0 / B)


@jax.jit
def kernel(logits, hidden_states):
    out = pl.pallas_call(
        _nce_body,
        grid=(B, NC),
        in_specs=[
            pl.BlockSpec((B, C), lambda b, c: (0, 0)),
            pl.BlockSpec((1, CH, L, D), lambda b, c: (b, c, 0, 0)),
        ],
        out_specs=pl.BlockSpec((1, 1), lambda b, c: (0, 0)),
        out_shape=jax.ShapeDtypeStruct((1, 1), jnp.float32),
        scratch_shapes=[
            pltpu.VMEM((1, D), jnp.float32),
            pltpu.VMEM((B, N), jnp.float32),
            pltpu.VMEM((N, B), jnp.float32),
        ],
    )(logits, hidden_states)
    return jnp.reshape(out, ())


# speedup vs baseline: 4.7132x; 1.2202x over previous
import jax
import jax.numpy as jnp
from jax.experimental import pallas as pl
from jax.experimental.pallas import tpu as pltpu

B = 16
C = 33
N = C - 1
L = 128
D = 768
K = 16
CH = 11
NC = C // CH
INV_T = 20.0
EPS = 1e-8


def _nce_body(logits_ref, h_ref, out_ref, pos_ref, sims_t_ref):
    b = pl.program_id(0)
    c = pl.program_id(1)

    x = h_ref[0]
    rows = L
    while rows > 8:
        half = rows // 2
        x = x[:, :half] + x[:, half:rows]
        rows = half
    means = jnp.sum(x, axis=1) * (1.0 / L)

    @pl.when(c == 0)
    def _store_pos():
        pos_ref[...] = means[0:1]

    p = pos_ref[...]
    na = jnp.maximum(jnp.sqrt(jnp.sum(p * p)), EPS)
    dots = jnp.sum(means * p, axis=1, keepdims=True)
    nb = jnp.maximum(
        jnp.sqrt(jnp.sum(means * means, axis=1, keepdims=True)), EPS)
    vals = dots / (na * nb)

    lane_b = jax.lax.broadcasted_iota(jnp.int32, (CH, B), 1)
    old = sims_t_ref[pl.ds(c * CH, CH), :]
    sims_t_ref[pl.ds(c * CH, CH), :] = jnp.where(lane_b == b, vals, old)

    @pl.when((b == B - 1) & (c == NC - 1))
    def _tail():
        t = sims_t_ref[1:C, :]
        ib = jax.lax.broadcasted_iota(jnp.int32, (B, B), 0)
        jb = jax.lax.broadcasted_iota(jnp.int32, (B, B), 1)
        eye = (ib == jb).astype(jnp.float32)
        u = jax.lax.dot_general(
            eye, t, (((1,), (1,)), ((), ())),
            preferred_element_type=jnp.float32)

        a = logits_ref[...] * INV_T
        ii = jax.lax.broadcasted_iota(jnp.int32, (N, N), 1)
        jj = jax.lax.broadcasted_iota(jnp.int32, (N, N), 0)
        loss_sum = jnp.zeros((1, 1), dtype=jnp.float32)
        for bi in range(B):
            s_lane = jnp.broadcast_to(u[bi:bi + 1, :], (N, N))
            s_sub = jnp.broadcast_to(t[:, bi:bi + 1], (N, N))
            beats = (s_sub > s_lane) | ((s_sub == s_lane) & (jj < ii))
            rank = jnp.sum(beats.astype(jnp.float32), axis=0, keepdims=True)
            sel = rank < K
            a_row = a[bi:bi + 1, :]
            m = jnp.max(a_row, axis=1, keepdims=True)
            e = jnp.exp(a_row - m)
            z = jnp.sum(e, axis=1, keepdims=True)
            pr = e / z
            p0 = pr[:, 0:1]
            neg_sum = jnp.sum(jnp.where(sel, pr[:, 1:], 0.0),
                              axis=1, keepdims=True)
            loss_sum = loss_sum - jnp.log(p0 / (p0 + neg_sum))
        out_ref[...] = loss_sum * (1.0 / B)


@jax.jit
def kernel(logits, hidden_states):
    out = pl.pallas_call(
        _nce_body,
        grid=(B, NC),
        in_specs=[
            pl.BlockSpec((B, C), lambda b, c: (0, 0)),
            pl.BlockSpec((1, CH, L, D), lambda b, c: (b, c, 0, 0)),
        ],
        out_specs=pl.BlockSpec((1, 1), lambda b, c: (0, 0)),
        out_shape=jax.ShapeDtypeStruct((1, 1), jnp.float32),
        scratch_shapes=[
            pltpu.VMEM((1, D), jnp.float32),
            pltpu.VMEM((C, B), jnp.float32),
        ],
    )(logits, hidden_states)
    return jnp.reshape(out, ())


# speedup vs baseline: 5.6785x; 1.2048x over previous
import jax
import jax.numpy as jnp
from jax.experimental import pallas as pl
from jax.experimental.pallas import tpu as pltpu

B = 16
C = 33
N = C - 1
L = 128
D = 768
K = 16
INV_T = 20.0
EPS = 1e-8


def _nce_body(logits_ref, h_ref, out_ref, acc_ref):
    b = pl.program_id(0)

    x = h_ref[0]
    rows = L
    while rows > 8:
        half = rows // 2
        x = x[:, :half] + x[:, half:rows]
        rows = half
    means = jnp.sum(x, axis=1) * (1.0 / L)

    p = means[0:1]
    na = jnp.maximum(jnp.sqrt(jnp.sum(p * p)), EPS)
    dots = jnp.sum(means * p, axis=1, keepdims=True)
    nb = jnp.maximum(
        jnp.sqrt(jnp.sum(means * means, axis=1, keepdims=True)), EPS)
    vals = dots / (na * nb)
    col = vals[1:C]

    inn = jax.lax.broadcasted_iota(jnp.int32, (N, N), 0)
    jnn = jax.lax.broadcasted_iota(jnp.int32, (N, N), 1)
    eye = (inn == jnn).astype(jnp.float32)
    row = jax.lax.dot_general(
        col, eye, (((0,), (0,)), ((), ())),
        preferred_element_type=jnp.float32)

    s_lane = jnp.broadcast_to(row, (N, N))
    s_sub = jnp.broadcast_to(col, (N, N))
    beats = (s_sub > s_lane) | ((s_sub == s_lane) & (inn < jnn))
    rank = jnp.sum(beats.astype(jnp.float32), axis=0, keepdims=True)
    sel = rank < K

    a_row = logits_ref[pl.ds(b, 1), :] * INV_T
    m = jnp.max(a_row, axis=1, keepdims=True)
    e = jnp.exp(a_row - m)
    z = jnp.sum(e, axis=1, keepdims=True)
    pr = e / z
    p0 = pr[:, 0:1]
    neg_sum = jnp.sum(jnp.where(sel, pr[:, 1:], 0.0),
                      axis=1, keepdims=True)
    loss_b = -jnp.log(p0 / (p0 + neg_sum))

    @pl.when(b == 0)
    def _init():
        acc_ref[...] = jnp.zeros((1, 1), jnp.float32)

    acc_ref[...] = acc_ref[...] + loss_b

    @pl.when(b == B - 1)
    def _emit():
        out_ref[...] = acc_ref[...] * (1.0 / B)


@jax.jit
def kernel(logits, hidden_states):
    out = pl.pallas_call(
        _nce_body,
        grid=(B,),
        in_specs=[
            pl.BlockSpec((B, C), lambda b: (0, 0)),
            pl.BlockSpec((1, C, L, D), lambda b: (b, 0, 0, 0)),
        ],
        out_specs=pl.BlockSpec((1, 1), lambda b: (0, 0)),
        out_shape=jax.ShapeDtypeStruct((1, 1), jnp.float32),
        scratch_shapes=[
            pltpu.VMEM((1, 1), jnp.float32),
        ],
    )(logits, hidden_states)
    return jnp.reshape(out, ())
